# baseline (speedup 1.0000x reference)
import jax
import jax.numpy as jnp
from jax.experimental import pallas as pl
from jax.experimental.pallas import tpu as pltpu


def _copy2_kernel(x_ref, a_ref, b_ref):
    v = x_ref[...]
    a_ref[...] = v
    b_ref[...] = v


def kernel(x, mask):
    B, S, D = x.shape
    SB = 1024
    blk = (1, SB, D)
    spec = pl.BlockSpec(blk, lambda i, j: (i, j, 0))
    out = pl.pallas_call(
        _copy2_kernel,
        grid=(B, S // SB),
        in_specs=[spec],
        out_specs=[spec, spec],
        out_shape=[
            jax.ShapeDtypeStruct(x.shape, x.dtype),
            jax.ShapeDtypeStruct(x.shape, x.dtype),
        ],
        compiler_params=pltpu.CompilerParams(
            dimension_semantics=("parallel", "parallel"),
        ),
    )(x)
    return (out[0], out[1])


# speedup vs baseline: 1.7491x; 1.7491x over previous
import jax
import jax.numpy as jnp
from jax.experimental import pallas as pl
from jax.experimental.pallas import tpu as pltpu


def _copy2_kernel(x_ref, a_ref, b_ref):
    v = x_ref[...]
    a_ref[...] = v
    b_ref[...] = v


def kernel(x, mask):
    B, S, D = x.shape
    BB = 2
    blk = (BB, S, D)
    spec = pl.BlockSpec(blk, lambda i: (i, 0, 0))
    out = pl.pallas_call(
        _copy2_kernel,
        grid=(B // BB,),
        in_specs=[spec],
        out_specs=[spec, spec],
        out_shape=[
            jax.ShapeDtypeStruct(x.shape, x.dtype),
            jax.ShapeDtypeStruct(x.shape, x.dtype),
        ],
        compiler_params=pltpu.CompilerParams(
            dimension_semantics=("parallel",),
        ),
    )(x)
    return (out[0], out[1])


# speedup vs baseline: 1.8341x; 1.0486x over previous
import jax
import jax.numpy as jnp
from jax.experimental import pallas as pl
from jax.experimental.pallas import tpu as pltpu


def _copy2_kernel(x_ref, a_ref, b_ref):
    v = x_ref[...]
    a_ref[...] = v
    b_ref[...] = v


def kernel(x, mask):
    B, S, D = x.shape
    BB = 4
    blk = (BB, S, D)
    spec = pl.BlockSpec(blk, lambda i: (i, 0, 0))
    out = pl.pallas_call(
        _copy2_kernel,
        grid=(B // BB,),
        in_specs=[spec],
        out_specs=[spec, spec],
        out_shape=[
            jax.ShapeDtypeStruct(x.shape, x.dtype),
            jax.ShapeDtypeStruct(x.shape, x.dtype),
        ],
        compiler_params=pltpu.CompilerParams(
            dimension_semantics=("parallel",),
        ),
    )(x)
    return (out[0], out[1])


# speedup vs baseline: 1.8413x; 1.0039x over previous
import jax
import jax.numpy as jnp
from jax.experimental import pallas as pl
from jax.experimental.pallas import tpu as pltpu


def _copy2_kernel(x_ref, a_ref, b_ref):
    v = x_ref[...]
    a_ref[...] = v
    b_ref[...] = v


def kernel(x, mask):
    B, S, D = x.shape
    BB = 4
    blk = (BB, S, D)
    spec = pl.BlockSpec(blk, lambda i: (i, 0, 0))
    out = pl.pallas_call(
        _copy2_kernel,
        grid=(B // BB,),
        in_specs=[spec],
        out_specs=[spec, spec],
        out_shape=[
            jax.ShapeDtypeStruct(x.shape, x.dtype),
            jax.ShapeDtypeStruct(x.shape, x.dtype),
        ],
        compiler_params=pltpu.CompilerParams(
            dimension_semantics=("parallel",),
            vmem_limit_bytes=128 * 1024 * 1024,
        ),
    )(x)
    return (out[0], out[1])
